# baseline (speedup 1.0000x reference)
import functools

import jax
import jax.numpy as jnp
from jax import lax
from jax.experimental import pallas as pl
from jax.experimental.pallas import tpu as pltpu
from jax.experimental.pallas import tpu_sc as plsc

NUM_CORES = 2
NUM_SUBCORES = 16
NUM_WORKERS = NUM_CORES * NUM_SUBCORES

LANES = 16
CHUNK = 8
NPE = 4
NX = 8
AHEAD = 4


def _pe_add_kernel(n_rows: int, d: int):
    rows_per_w = n_rows // NUM_WORKERS
    n_chunks = rows_per_w // CHUNK
    assert n_chunks % NX == 0 and n_chunks >= 2 * NX
    mesh = plsc.VectorSubcoreMesh(core_axis_name="c", subcore_axis_name="s")

    @functools.partial(
        pl.kernel,
        mesh=mesh,
        out_type=jax.ShapeDtypeStruct((n_rows, d), jnp.float32),
        scratch_types=[
            [pltpu.VMEM((CHUNK,), jnp.int32) for _ in range(NPE)],
            [pltpu.VMEM((CHUNK, d), jnp.float32) for _ in range(NPE)],
            [pltpu.VMEM((CHUNK, d), jnp.float32) for _ in range(NX)],
            [pltpu.SemaphoreType.DMA for _ in range(NPE)],
            [pltpu.SemaphoreType.DMA for _ in range(NX)],
            [pltpu.SemaphoreType.DMA for _ in range(NX)],
        ],
    )
    def body(x_hbm, idx_hbm, pe_hbm, out_hbm,
             idx_v, pe_buf, x_buf, sem_g, sem_x, sem_o):
        wid = lax.axis_index("s") * NUM_CORES + lax.axis_index("c")
        base0 = wid * rows_per_w

        def issue_inputs(j, bp, bx):
            base = base0 + j * CHUNK
            pltpu.sync_copy(idx_hbm.at[pl.ds(base, CHUNK)], idx_v[bp])
            pltpu.async_copy(pe_hbm.at[idx_v[bp]], pe_buf[bp], sem_g[bp])
            pltpu.async_copy(x_hbm.at[pl.ds(base, CHUNK)], x_buf[bx], sem_x[bx])

        for b in range(AHEAD):
            issue_inputs(b, b % NPE, b % NX)

        @pl.loop(0, n_chunks, step=NX)
        def chunk_group(g):
            for b in range(NX):
                bp = b % NPE
                j = g + b
                base = base0 + j * CHUNK
                pltpu.make_async_copy(pe_hbm.at[idx_v[bp]], pe_buf[bp],
                                      sem_g[bp]).wait()
                pltpu.make_async_copy(x_hbm.at[pl.ds(base, CHUNK)],
                                      x_buf[b], sem_x[b]).wait()

                @plsc.parallel_loop(0, CHUNK)
                def row_body(r):
                    for c in range(d // LANES):
                        sl = pl.ds(c * LANES, LANES)
                        plsc.addupdate(x_buf[b].at[r, sl], pe_buf[bp][r, sl])

                pltpu.async_copy(x_buf[b], out_hbm.at[pl.ds(base, CHUNK)],
                                 sem_o[b])

                b2 = (b + AHEAD) % NX

                @pl.when(j >= AHEAD)
                def _():
                    pltpu.make_async_copy(
                        x_buf[b2],
                        out_hbm.at[pl.ds(base - AHEAD * CHUNK, CHUNK)],
                        sem_o[b2]).wait()

                @pl.when(j + AHEAD < n_chunks)
                def _():
                    issue_inputs(j + AHEAD, bp, b2)

        for j in range(n_chunks - AHEAD, n_chunks):
            base = base0 + j * CHUNK
            pltpu.make_async_copy(x_buf[j % NX],
                                  out_hbm.at[pl.ds(base, CHUNK)],
                                  sem_o[j % NX]).wait()

    return body


def kernel(x, time_fra, frame_emb, pe):
    b, s, d = x.shape
    n = b * s
    xf = x.reshape(n, d)
    idx = time_fra.reshape(n).astype(jnp.int32)
    out = _pe_add_kernel(n, d)(xf, idx, pe)
    return out.reshape(b, s, d)


# speedup vs baseline: 1.1081x; 1.1081x over previous
import functools

import jax
import jax.numpy as jnp
from jax import lax
from jax.experimental import pallas as pl
from jax.experimental.pallas import tpu as pltpu
from jax.experimental.pallas import tpu_sc as plsc

NUM_CORES = 2
NUM_SUBCORES = 16
NUM_WORKERS = NUM_CORES * NUM_SUBCORES

LANES = 16
CHUNK = 16
NPE = 2
NX = 4


def _pe_add_kernel(n_rows: int, d: int):
    rows_per_w = n_rows // NUM_WORKERS
    n_chunks = rows_per_w // CHUNK
    assert n_chunks % NX == 0 and n_chunks >= 2 * NX
    mesh = plsc.VectorSubcoreMesh(core_axis_name="c", subcore_axis_name="s")

    @functools.partial(
        pl.kernel,
        mesh=mesh,
        out_type=jax.ShapeDtypeStruct((n_rows, d), jnp.float32),
        scratch_types=[
            [pltpu.VMEM((CHUNK,), jnp.int32) for _ in range(NPE)],
            [pltpu.VMEM((CHUNK, d), jnp.float32) for _ in range(NPE)],
            [pltpu.VMEM((CHUNK, d), jnp.float32) for _ in range(NX)],
            [pltpu.SemaphoreType.DMA for _ in range(NPE)],
            [pltpu.SemaphoreType.DMA for _ in range(NPE)],
            [pltpu.SemaphoreType.DMA for _ in range(NX)],
            [pltpu.SemaphoreType.DMA for _ in range(NX)],
        ],
    )
    def body(x_hbm, idx_hbm, pe_hbm, out_hbm,
             idx_v, pe_buf, x_buf, sem_i, sem_g, sem_x, sem_o):
        wid = lax.axis_index("s") * NUM_CORES + lax.axis_index("c")
        base0 = wid * rows_per_w

        def idx_copy(j, bp):
            base = base0 + j * CHUNK
            return pltpu.make_async_copy(idx_hbm.at[pl.ds(base, CHUNK)],
                                         idx_v[bp], sem_i[bp])

        def issue_gather_x(j, bp, bx):
            base = base0 + j * CHUNK
            pltpu.async_copy(pe_hbm.at[idx_v[bp]], pe_buf[bp], sem_g[bp])
            pltpu.async_copy(x_hbm.at[pl.ds(base, CHUNK)], x_buf[bx], sem_x[bx])

        for b in range(NPE):
            idx_copy(b, b).start()
            idx_copy(b, b).wait()
            issue_gather_x(b, b, b)

        @pl.loop(0, n_chunks, step=NX)
        def chunk_group(g):
            for b in range(NX):
                bp = b % NPE
                j = g + b
                base = base0 + j * CHUNK
                pltpu.make_async_copy(pe_hbm.at[idx_v[bp]], pe_buf[bp],
                                      sem_g[bp]).wait()
                pltpu.make_async_copy(x_hbm.at[pl.ds(base, CHUNK)],
                                      x_buf[b], sem_x[b]).wait()

                @pl.when(j + 2 < n_chunks)
                def _():
                    idx_copy(j + 2, bp).start()

                @plsc.parallel_loop(0, CHUNK)
                def row_body(r):
                    for c in range(d // LANES):
                        sl = pl.ds(c * LANES, LANES)
                        plsc.addupdate(x_buf[b].at[r, sl], pe_buf[bp][r, sl])

                pltpu.async_copy(x_buf[b], out_hbm.at[pl.ds(base, CHUNK)],
                                 sem_o[b])

                b2 = (b + 2) % NX

                @pl.when(j >= 2)
                def _():
                    pltpu.make_async_copy(
                        x_buf[b2],
                        out_hbm.at[pl.ds(base - 2 * CHUNK, CHUNK)],
                        sem_o[b2]).wait()

                @pl.when(j + 2 < n_chunks)
                def _():
                    idx_copy(j + 2, bp).wait()
                    issue_gather_x(j + 2, bp, b2)

        for j in (n_chunks - 2, n_chunks - 1):
            base = base0 + j * CHUNK
            pltpu.make_async_copy(x_buf[j % NX],
                                  out_hbm.at[pl.ds(base, CHUNK)],
                                  sem_o[j % NX]).wait()

    return body


def kernel(x, time_fra, frame_emb, pe):
    b, s, d = x.shape
    n = b * s
    xf = x.reshape(n, d)
    idx = time_fra.reshape(n).astype(jnp.int32)
    out = _pe_add_kernel(n, d)(xf, idx, pe)
    return out.reshape(b, s, d)


# speedup vs baseline: 1.1139x; 1.0052x over previous
import functools

import jax
import jax.numpy as jnp
from jax import lax
from jax.experimental import pallas as pl
from jax.experimental.pallas import tpu as pltpu
from jax.experimental.pallas import tpu_sc as plsc

NUM_CORES = 2
NUM_SUBCORES = 16
NUM_WORKERS = NUM_CORES * NUM_SUBCORES

LANES = 16
CHUNK = 16
NPE = 2
NX = 4


def _pe_add_kernel(n_rows: int, d: int):
    rows_per_w = n_rows // NUM_WORKERS
    n_chunks = rows_per_w // CHUNK
    assert n_chunks % NX == 0 and n_chunks >= 2 * NX
    mesh = plsc.VectorSubcoreMesh(core_axis_name="c", subcore_axis_name="s")

    @functools.partial(
        pl.kernel,
        mesh=mesh,
        out_type=jax.ShapeDtypeStruct((n_rows, d), jnp.float32),
        scratch_types=[
            [pltpu.VMEM((CHUNK,), jnp.int32) for _ in range(NPE)],
            [pltpu.VMEM((CHUNK, d), jnp.float32) for _ in range(NPE)],
            [pltpu.VMEM((CHUNK, d), jnp.float32) for _ in range(NX)],
            [pltpu.SemaphoreType.DMA for _ in range(NPE)],
            [pltpu.SemaphoreType.DMA for _ in range(NPE)],
            [pltpu.SemaphoreType.DMA for _ in range(NX)],
            [pltpu.SemaphoreType.DMA for _ in range(NX)],
        ],
    )
    def body(x_hbm, idx_hbm, pe_hbm, out_hbm,
             idx_v, pe_buf, x_buf, sem_i, sem_g, sem_x, sem_o):
        wid = lax.axis_index("s") * NUM_CORES + lax.axis_index("c")
        base0 = wid * rows_per_w

        def idx_copy(j, bp):
            base = base0 + j * CHUNK
            return pltpu.make_async_copy(idx_hbm.at[pl.ds(base, CHUNK)],
                                         idx_v[bp], sem_i[bp])

        def issue_gather_x(j, bp, bx):
            base = base0 + j * CHUNK
            pltpu.async_copy(pe_hbm.at[idx_v[bp]], pe_buf[bp], sem_g[bp])
            pltpu.async_copy(x_hbm.at[pl.ds(base, CHUNK)], x_buf[bx], sem_x[bx])

        for b in range(NPE):
            idx_copy(b, b).start()
            idx_copy(b, b).wait()
            issue_gather_x(b, b, b)

        @pl.loop(0, n_chunks, step=NX)
        def chunk_group(g):
            for b in range(NX):
                bp = b % NPE
                j = g + b
                base = base0 + j * CHUNK
                pltpu.make_async_copy(pe_hbm.at[idx_v[bp]], pe_buf[bp],
                                      sem_g[bp]).wait()
                pltpu.make_async_copy(x_hbm.at[pl.ds(base, CHUNK)],
                                      x_buf[b], sem_x[b]).wait()

                @pl.when(j + 2 < n_chunks)
                def _():
                    idx_copy(j + 2, bp).start()

                b2 = (b + 2) % NX
                base2 = base + 2 * CHUNK

                @pl.when(j >= 2)
                def _():
                    pltpu.make_async_copy(
                        x_buf[b2],
                        out_hbm.at[pl.ds(base - 2 * CHUNK, CHUNK)],
                        sem_o[b2]).wait()

                @pl.when(j + 2 < n_chunks)
                def _():
                    pltpu.async_copy(x_hbm.at[pl.ds(base2, CHUNK)],
                                     x_buf[b2], sem_x[b2])

                @plsc.parallel_loop(0, CHUNK)
                def row_body(r):
                    for c in range(d // LANES):
                        sl = pl.ds(c * LANES, LANES)
                        plsc.addupdate(x_buf[b].at[r, sl], pe_buf[bp][r, sl])

                pltpu.async_copy(x_buf[b], out_hbm.at[pl.ds(base, CHUNK)],
                                 sem_o[b])

                @pl.when(j + 2 < n_chunks)
                def _():
                    idx_copy(j + 2, bp).wait()
                    pltpu.async_copy(pe_hbm.at[idx_v[bp]], pe_buf[bp],
                                     sem_g[bp])

        for j in (n_chunks - 2, n_chunks - 1):
            base = base0 + j * CHUNK
            pltpu.make_async_copy(x_buf[j % NX],
                                  out_hbm.at[pl.ds(base, CHUNK)],
                                  sem_o[j % NX]).wait()

    return body


def kernel(x, time_fra, frame_emb, pe):
    b, s, d = x.shape
    n = b * s
    xf = x.reshape(n, d)
    idx = time_fra.reshape(n).astype(jnp.int32)
    out = _pe_add_kernel(n, d)(xf, idx, pe)
    return out.reshape(b, s, d)


# speedup vs baseline: 1.1390x; 1.0226x over previous
import functools

import jax
import jax.numpy as jnp
from jax import lax
from jax.experimental import pallas as pl
from jax.experimental.pallas import tpu as pltpu
from jax.experimental.pallas import tpu_sc as plsc

NUM_CORES = 2
NUM_SUBCORES = 16
NUM_WORKERS = NUM_CORES * NUM_SUBCORES

LANES = 16
CHUNK = 16
NPE = 2
NX = 4


def _pe_add_kernel(n_rows: int, d: int):
    rows_per_w = n_rows // NUM_WORKERS
    n_chunks = rows_per_w // CHUNK
    assert n_chunks % NX == 0 and n_chunks >= 2 * NX
    mesh = plsc.VectorSubcoreMesh(core_axis_name="c", subcore_axis_name="s")

    @functools.partial(
        pl.kernel,
        mesh=mesh,
        out_type=jax.ShapeDtypeStruct((n_rows, d), jnp.float32),
        scratch_types=[
            [pltpu.VMEM((CHUNK,), jnp.int32) for _ in range(NPE)],
            [pltpu.VMEM((CHUNK, d), jnp.float32) for _ in range(NPE)],
            [pltpu.VMEM((CHUNK, d), jnp.float32) for _ in range(NX)],
            [pltpu.SemaphoreType.DMA for _ in range(NPE)],
            [pltpu.SemaphoreType.DMA for _ in range(NPE)],
            [pltpu.SemaphoreType.DMA for _ in range(NX)],
            [pltpu.SemaphoreType.DMA for _ in range(NX)],
        ],
    )
    def body(x_hbm, idx_hbm, pe_hbm, out_hbm,
             idx_v, pe_buf, x_buf, sem_i, sem_g, sem_x, sem_o):
        wid = lax.axis_index("s") * NUM_CORES + lax.axis_index("c")
        base0 = wid * rows_per_w

        def idx_copy(j, bp):
            base = base0 + j * CHUNK
            return pltpu.make_async_copy(idx_hbm.at[pl.ds(base, CHUNK)],
                                         idx_v[bp], sem_i[bp])

        def issue_gather_x(j, bp, bx):
            base = base0 + j * CHUNK
            pltpu.async_copy(pe_hbm.at[idx_v[bp]], pe_buf[bp], sem_g[bp])
            pltpu.async_copy(x_hbm.at[pl.ds(base, CHUNK)], x_buf[bx], sem_x[bx])

        for b in range(NPE):
            idx_copy(b, b).start()
            idx_copy(b, b).wait()
            issue_gather_x(b, b, b)

        @pl.loop(0, n_chunks, step=NX)
        def chunk_group(g):
            for b in range(NX):
                bp = b % NPE
                j = g + b
                base = base0 + j * CHUNK
                pltpu.make_async_copy(pe_hbm.at[idx_v[bp]], pe_buf[bp],
                                      sem_g[bp]).wait()
                pltpu.make_async_copy(x_hbm.at[pl.ds(base, CHUNK)],
                                      x_buf[b], sem_x[b]).wait()

                @pl.when(j + 2 < n_chunks)
                def _():
                    idx_copy(j + 2, bp).start()

                b2 = (b + 2) % NX
                base2 = base + 2 * CHUNK

                @pl.when(j >= 2)
                def _():
                    pltpu.make_async_copy(
                        x_buf[b2],
                        out_hbm.at[pl.ds(base - 2 * CHUNK, CHUNK)],
                        sem_o[b2]).wait()

                @pl.when(j + 2 < n_chunks)
                def _():
                    pltpu.async_copy(x_hbm.at[pl.ds(base2, CHUNK)],
                                     x_buf[b2], sem_x[b2])

                cpr = d // LANES
                shift = cpr.bit_length() - 1

                @plsc.parallel_loop(0, CHUNK * cpr, unroll=8)
                def sl_body(i):
                    r = lax.shift_right_logical(i, shift)
                    sl = pl.ds((i & (cpr - 1)) * LANES, LANES)
                    plsc.addupdate(x_buf[b].at[r, sl], pe_buf[bp][r, sl])

                pltpu.async_copy(x_buf[b], out_hbm.at[pl.ds(base, CHUNK)],
                                 sem_o[b])

                @pl.when(j + 2 < n_chunks)
                def _():
                    idx_copy(j + 2, bp).wait()
                    pltpu.async_copy(pe_hbm.at[idx_v[bp]], pe_buf[bp],
                                     sem_g[bp])

        for j in (n_chunks - 2, n_chunks - 1):
            base = base0 + j * CHUNK
            pltpu.make_async_copy(x_buf[j % NX],
                                  out_hbm.at[pl.ds(base, CHUNK)],
                                  sem_o[j % NX]).wait()

    return body


def kernel(x, time_fra, frame_emb, pe):
    b, s, d = x.shape
    n = b * s
    xf = x.reshape(n, d)
    idx = time_fra.reshape(n).astype(jnp.int32)
    out = _pe_add_kernel(n, d)(xf, idx, pe)
    return out.reshape(b, s, d)
